# Initial kernel scaffold; baseline (speedup 1.0000x reference)
import jax
import jax.numpy as jnp
from jax.experimental import pallas as pl


def kernel(x, proj1, proj1_bias, proj2, proj2_bias, gate_w):
    raise NotImplementedError("write your pallas kernel here")


# speedup vs baseline: 1.9723x; 1.9723x over previous
import jax
import jax.numpy as jnp
from jax.experimental import pallas as pl
from jax.experimental.pallas import tpu as pltpu

_EMBED = 2048
_HIDDEN = 8192
_BM = 512
_BH = 2048

_INV_SQRT2 = 0.7071067811865476


def _mlp_body(x_ref, w1_ref, b1_ref, w2_ref, b2_ref, o_ref):
    j = pl.program_id(1)
    h = jax.lax.dot_general(
        x_ref[...], w1_ref[...],
        (((1,), (1,)), ((), ())),
        preferred_element_type=jnp.float32)
    h = h + b1_ref[...]
    h = 0.5 * h * (1.0 + jax.lax.erf(h * _INV_SQRT2))
    contrib = jax.lax.dot_general(
        h.astype(jnp.bfloat16), w2_ref[...],
        (((1,), (1,)), ((), ())),
        preferred_element_type=jnp.float32)

    @pl.when(j == 0)
    def _init():
        o_ref[...] = contrib + b2_ref[...]

    @pl.when(j != 0)
    def _acc():
        o_ref[...] += contrib


def kernel(x, proj1, proj1_bias, proj2, proj2_bias, gate_w):
    del gate_w
    L, N, E = x.shape
    M = L * N
    xb = x.reshape(M, E).astype(jnp.bfloat16)
    w1 = proj1.astype(jnp.bfloat16)
    w2 = proj2.astype(jnp.bfloat16)
    b1 = proj1_bias.reshape(1, _HIDDEN)
    b2 = proj2_bias.reshape(1, _EMBED)

    grid = (M // _BM, _HIDDEN // _BH)
    out = pl.pallas_call(
        _mlp_body,
        grid=grid,
        in_specs=[
            pl.BlockSpec((_BM, _EMBED), lambda i, j: (i, 0)),
            pl.BlockSpec((_BH, _EMBED), lambda i, j: (j, 0)),
            pl.BlockSpec((1, _BH), lambda i, j: (0, j)),
            pl.BlockSpec((_EMBED, _BH), lambda i, j: (0, j)),
            pl.BlockSpec((1, _EMBED), lambda i, j: (0, 0)),
        ],
        out_specs=pl.BlockSpec((_BM, _EMBED), lambda i, j: (i, 0)),
        out_shape=jax.ShapeDtypeStruct((M, E), jnp.float32),
        compiler_params=pltpu.CompilerParams(
            dimension_semantics=("parallel", "arbitrary"),
        ),
    )(xb, w1, b1, w2, b2)
    return out.reshape(L, N, E)
